# baseline (device time: 11879 ns/iter reference)
import jax
import jax.numpy as jnp
from jax import lax
from jax.experimental import pallas as pl
from jax.experimental.pallas import tpu as pltpu


N_CHUNK = 4


def kernel(x):
    m, n = x.shape
    half = n // 2
    rpc = m // N_CHUNK

    def body(x_ref, out_ref, send_buf, send_sems, recv_sems):
        my_x = lax.axis_index("x")
        my_y = lax.axis_index("y")
        my_z = lax.axis_index("z")
        peer_x = 1 - my_x
        peer = (peer_x, my_y, my_z)

        barrier_sem = pltpu.get_barrier_semaphore()
        pl.semaphore_signal(
            barrier_sem, inc=1,
            device_id=peer, device_id_type=pl.DeviceIdType.MESH,
        )
        pl.semaphore_wait(barrier_sem, 1)

        rdmas = []
        for c in range(N_CHUNK):
            send_buf[pl.ds(c * rpc, rpc), :] = x_ref[
                pl.ds(c * rpc, rpc), pl.ds(peer_x * half, half)
            ].astype(jnp.bfloat16)
            rdma = pltpu.make_async_remote_copy(
                src_ref=send_buf.at[pl.ds(c * rpc, rpc), :],
                dst_ref=out_ref.at[pl.ds(my_x * m + c * rpc, rpc), :],
                send_sem=send_sems.at[c],
                recv_sem=recv_sems.at[c],
                device_id=peer,
                device_id_type=pl.DeviceIdType.MESH,
            )
            rdma.start()
            rdmas.append(rdma)

        out_ref[pl.ds(my_x * m, m), :] = x_ref[:, pl.ds(my_x * half, half)].astype(
            jnp.bfloat16
        )

        for rdma in rdmas:
            rdma.wait()

    return pl.pallas_call(
        body,
        out_shape=jax.ShapeDtypeStruct((2 * m, half), jnp.bfloat16),
        in_specs=[pl.BlockSpec(memory_space=pltpu.VMEM)],
        out_specs=pl.BlockSpec(memory_space=pltpu.VMEM),
        scratch_shapes=[
            pltpu.VMEM((m, half), jnp.bfloat16),
            pltpu.SemaphoreType.DMA((N_CHUNK,)),
            pltpu.SemaphoreType.DMA((N_CHUNK,)),
        ],
        compiler_params=pltpu.CompilerParams(collective_id=0),
    )(x)
